# baseline (speedup 1.0000x reference)
import functools

import jax
import jax.numpy as jnp
import numpy as np
from jax.experimental import pallas as pl

_P8 = np.kron(np.eye(8, dtype=np.float32),
              np.pad(np.eye(32, dtype=np.float32), ((0, 0), (0, 32))))


def _fm_block(attrs_ref, emb_ref, w8_ref, p8_ref, out_ref):
    out_ref[...] = (
        jnp.dot(emb_ref[...], p8_ref[...], preferred_element_type=jnp.float32)
        + jnp.dot(attrs_ref[...], w8_ref[...], preferred_element_type=jnp.float32)
    )


@functools.partial(jax.jit, static_argnames=("block_flat_rows",))
def _fm(attrs, emb, w, block_flat_rows):
    n = emb.shape[0]
    nf = n // 8
    a8 = attrs.reshape(nf, 128)
    e8 = emb.reshape(nf, 256)
    w8 = jnp.kron(jnp.eye(8, dtype=w.dtype), jnp.pad(w, ((0, 0), (32, 0))))
    grid = (pl.cdiv(nf, block_flat_rows),)
    out = pl.pallas_call(
        _fm_block,
        grid=grid,
        in_specs=[
            pl.BlockSpec((block_flat_rows, 128), lambda i: (i, 0)),
            pl.BlockSpec((block_flat_rows, 256), lambda i: (i, 0)),
            pl.BlockSpec((128, 512), lambda i: (0, 0)),
            pl.BlockSpec((256, 512), lambda i: (0, 0)),
        ],
        out_specs=pl.BlockSpec((block_flat_rows, 512), lambda i: (i, 0)),
        out_shape=jax.ShapeDtypeStruct((nf, 512), jnp.float32),
    )(a8, e8, w8, jnp.asarray(_P8))
    return out.reshape(n, 64)


def kernel(user_attrs_input, item_attrs_input, user_emb, item_emb,
           user_attrs_trans_w, item_attrs_trans_w):
    final_user = _fm(user_attrs_input, user_emb, user_attrs_trans_w, 1000)
    final_item = _fm(item_attrs_input, item_emb, item_attrs_trans_w, 1000)
    return (final_user, final_item)


# speedup vs baseline: 1.1437x; 1.1437x over previous
import functools

import jax
import jax.numpy as jnp
from jax.experimental import pallas as pl


def _fm_block(attrs_ref, emb_ref, w_ref, out_ref):
    ae = jnp.dot(attrs_ref[...], w_ref[...], preferred_element_type=jnp.float32)
    out_ref[...] = jnp.concatenate([emb_ref[...], ae], axis=1)


@functools.partial(jax.jit, static_argnames=("block_rows",))
def _fm(attrs, emb, w, block_rows):
    n, d_emb = emb.shape
    d_attr = attrs.shape[1]
    d_out = d_emb + w.shape[1]
    grid = (pl.cdiv(n, block_rows),)
    return pl.pallas_call(
        _fm_block,
        grid=grid,
        in_specs=[
            pl.BlockSpec((block_rows, d_attr), lambda i: (i, 0)),
            pl.BlockSpec((block_rows, d_emb), lambda i: (i, 0)),
            pl.BlockSpec((d_attr, w.shape[1]), lambda i: (0, 0)),
        ],
        out_specs=pl.BlockSpec((block_rows, d_out), lambda i: (i, 0)),
        out_shape=jax.ShapeDtypeStruct((n, d_out), jnp.float32),
    )(attrs, emb, w)


def kernel(user_attrs_input, item_attrs_input, user_emb, item_emb,
           user_attrs_trans_w, item_attrs_trans_w):
    final_user = _fm(user_attrs_input, user_emb, user_attrs_trans_w, 10000)
    final_item = _fm(item_attrs_input, item_emb, item_attrs_trans_w, 10000)
    return (final_user, final_item)


# speedup vs baseline: 10.1860x; 8.9064x over previous
import functools

import jax
import jax.numpy as jnp
from jax.experimental import pallas as pl


def _fm_block(attrs_t_ref, emb_t_ref, w_t_ref, out_t_ref):
    out_t_ref[0:32, :] = emb_t_ref[...]
    out_t_ref[32:64, :] = jnp.dot(w_t_ref[...], attrs_t_ref[...],
                                  preferred_element_type=jnp.float32)


@functools.partial(jax.jit, static_argnames=("block_cols",))
def _fm(attrs, emb, w, block_cols):
    n, d_emb = emb.shape
    d_attr = attrs.shape[1]
    d_out = d_emb + w.shape[1]
    attrs_t = attrs.T
    emb_t = emb.T
    w_t = w.T
    grid = (pl.cdiv(n, block_cols),)
    out_t = pl.pallas_call(
        _fm_block,
        grid=grid,
        in_specs=[
            pl.BlockSpec((d_attr, block_cols), lambda i: (0, i)),
            pl.BlockSpec((d_emb, block_cols), lambda i: (0, i)),
            pl.BlockSpec((w.shape[1], d_attr), lambda i: (0, 0)),
        ],
        out_specs=pl.BlockSpec((d_out, block_cols), lambda i: (0, i)),
        out_shape=jax.ShapeDtypeStruct((d_out, n), jnp.float32),
    )(attrs_t, emb_t, w_t)
    return out_t.T


def kernel(user_attrs_input, item_attrs_input, user_emb, item_emb,
           user_attrs_trans_w, item_attrs_trans_w):
    final_user = _fm(user_attrs_input, user_emb, user_attrs_trans_w, 16384)
    final_item = _fm(item_attrs_input, item_emb, item_attrs_trans_w, 16384)
    return (final_user, final_item)


# speedup vs baseline: 10.6788x; 1.0484x over previous
import functools

import jax
import jax.numpy as jnp
from jax.experimental import pallas as pl


def _fm_block(attrs_t_ref, emb_t_ref, w_t_ref, out_t_ref):
    out_t_ref[0:32, :] = emb_t_ref[...]
    out_t_ref[32:64, :] = jnp.dot(w_t_ref[...], attrs_t_ref[...],
                                  preferred_element_type=jnp.float32)


@functools.partial(jax.jit, static_argnames=("block_cols",))
def _fm(attrs, emb, w, block_cols):
    n, d_emb = emb.shape
    d_attr = attrs.shape[1]
    d_out = d_emb + w.shape[1]
    attrs_t = attrs.T
    emb_t = emb.T
    w_t = w.T
    grid = (pl.cdiv(n, block_cols),)
    out_t = pl.pallas_call(
        _fm_block,
        grid=grid,
        in_specs=[
            pl.BlockSpec((d_attr, block_cols), lambda i: (0, i)),
            pl.BlockSpec((d_emb, block_cols), lambda i: (0, i)),
            pl.BlockSpec((w.shape[1], d_attr), lambda i: (0, 0)),
        ],
        out_specs=pl.BlockSpec((d_out, block_cols), lambda i: (0, i)),
        out_shape=jax.ShapeDtypeStruct((d_out, n), jnp.float32),
    )(attrs_t, emb_t, w_t)
    return out_t.T


def kernel(user_attrs_input, item_attrs_input, user_emb, item_emb,
           user_attrs_trans_w, item_attrs_trans_w):
    final_user = _fm(user_attrs_input, user_emb, user_attrs_trans_w, 32768)
    final_item = _fm(item_attrs_input, item_emb, item_attrs_trans_w, 32768)
    return (final_user, final_item)


# speedup vs baseline: 10.7848x; 1.0099x over previous
import functools

import jax
import jax.numpy as jnp
from jax.experimental import pallas as pl


def _fm_block(attrs_t_ref, emb_t_ref, w_t_ref, out_t_ref):
    out_t_ref[0:32, :] = emb_t_ref[...]
    out_t_ref[32:64, :] = jnp.dot(w_t_ref[...], attrs_t_ref[...],
                                  preferred_element_type=jnp.float32)


@functools.partial(jax.jit, static_argnames=("block_cols",))
def _fm(attrs, emb, w, block_cols):
    n, d_emb = emb.shape
    d_attr = attrs.shape[1]
    d_out = d_emb + w.shape[1]
    attrs_t = attrs.T
    emb_t = emb.T
    w_t = w.T
    grid = (pl.cdiv(n, block_cols),)
    out_t = pl.pallas_call(
        _fm_block,
        grid=grid,
        in_specs=[
            pl.BlockSpec((d_attr, block_cols), lambda i: (0, i)),
            pl.BlockSpec((d_emb, block_cols), lambda i: (0, i)),
            pl.BlockSpec((w.shape[1], d_attr), lambda i: (0, 0)),
        ],
        out_specs=pl.BlockSpec((d_out, block_cols), lambda i: (0, i)),
        out_shape=jax.ShapeDtypeStruct((d_out, n), jnp.float32),
    )(attrs_t, emb_t, w_t)
    return out_t.T


def kernel(user_attrs_input, item_attrs_input, user_emb, item_emb,
           user_attrs_trans_w, item_attrs_trans_w):
    final_user = _fm(user_attrs_input, user_emb, user_attrs_trans_w, 65536)
    final_item = _fm(item_attrs_input, item_emb, item_attrs_trans_w, 65536)
    return (final_user, final_item)
